# baseline (speedup 1.0000x reference)
import functools

import jax
import jax.numpy as jnp
from jax import lax
from jax.experimental import pallas as pl
from jax.experimental.pallas import tpu as pltpu
from jax.experimental.pallas import tpu_sc as plsc

_VOCAB = 1000000
_D = 64
_B = 16384
_K = 20
_NC = 2
_NS = 16
_NW = _NC * _NS
_BPW = _B // _NW
_CH = 32
_NR = _BPW // _CH
_NCHUNK = _NW * _NR
_NEG_CH = _CH * _K
_NIDX_ROWS = _NEG_CH // 128
_POS_ROWS = _B * 16 // 128
_NEG_ROWS = _B * _K * 16 // 128


def _full16(v):
    return jnp.full((16,), v, jnp.int32)


_CPR = 400
_NCHK = _VOCAB // _CPR


def _sc_compact(table):
    mesh = plsc.VectorSubcoreMesh(core_axis_name="c", subcore_axis_name="s")
    per_w = (_NCHK + _NW - 1) // _NW

    @functools.partial(
        pl.kernel,
        mesh=mesh,
        out_type=jax.ShapeDtypeStruct((_VOCAB // 2, 128), jnp.float32),
        scratch_types=[
            pltpu.VMEM((2 * _CPR, _D), jnp.float32),
            pltpu.VMEM((_CPR // 2, 128), jnp.float32),
            pltpu.SemaphoreType.DMA,
            pltpu.SemaphoreType.DMA,
        ],
    )
    def k(tbl_hbm, out_hbm, bufa, bufb, sem_in, sem_out):
        wid = lax.axis_index("s") * _NC + lax.axis_index("c")

        def in_copy(i):
            c = wid + _NW * i
            boff = pl.multiple_of((i % 2) * _CPR, 8)
            return pltpu.async_copy(
                tbl_hbm.at[pl.ds(pl.multiple_of(c * _CPR, 8), _CPR)],
                bufa.at[pl.ds(boff, _CPR)], sem_in)

        @pl.when(wid < _NCHK)
        def _():
            in_copy(0)

        def chunk_body(i, carry):
            c = wid + _NW * i

            @pl.when(c < _NCHK)
            def _():
                @pl.when(c + _NW < _NCHK)
                def _():
                    in_copy(i + 1)

                pltpu.make_async_copy(
                    tbl_hbm.at[pl.ds(pl.multiple_of(c * _CPR, 8), _CPR)],
                    bufa.at[pl.ds(pl.multiple_of((i % 2) * _CPR, 8), _CPR)],
                    sem_in).wait()

                @pl.when(i > 0)
                def _():
                    pltpu.make_async_copy(
                        bufb,
                        out_hbm.at[pl.ds(0, _CPR // 2)], sem_out).wait()

                boff = pl.multiple_of((i % 2) * _CPR, 8)

                def row_body(j, carry2):
                    r0 = boff + 2 * j
                    for kq in range(4):
                        bufb[j, pl.ds(kq * 16, 16)] = (
                            bufa[r0, pl.ds(kq * 16, 16)])
                        bufb[j, pl.ds(64 + kq * 16, 16)] = (
                            bufa[r0 + 1, pl.ds(kq * 16, 16)])
                    return carry2

                lax.fori_loop(0, _CPR // 2, row_body, 0)
                pltpu.async_copy(
                    bufb,
                    out_hbm.at[pl.ds(pl.multiple_of(c * (_CPR // 2), 8),
                                     _CPR // 2)], sem_out)

            return carry

        lax.fori_loop(0, per_w, chunk_body, 0)

        @pl.when(wid < _NCHK)
        def _():
            pltpu.make_async_copy(
                bufb, out_hbm.at[pl.ds(0, _CPR // 2)], sem_out).wait()

    return k(table)


def _sc_scores(cc, negio, ine2, oute2):
    mesh = plsc.VectorSubcoreMesh(core_axis_name="c", subcore_axis_name="s")

    @functools.partial(
        pl.kernel,
        mesh=mesh,
        out_type=[
            jax.ShapeDtypeStruct((_POS_ROWS, 128), jnp.float32),
            jax.ShapeDtypeStruct((_NEG_ROWS, 128), jnp.float32),
        ],
        scratch_types=[
            pltpu.VMEM((_NR * 128,), jnp.int32),
            pltpu.VMEM((2 * 2 * _NEG_CH + 16,), jnp.int32),
            pltpu.VMEM((_CH, 128), jnp.float32),
            pltpu.VMEM((_CH, 128), jnp.float32),
            pltpu.VMEM((_NEG_CH, 128), jnp.float32),
            pltpu.VMEM((_BPW // 8, 128), jnp.float32),
            pltpu.VMEM((_NEG_CH // 8, 128), jnp.float32),
            pltpu.SemaphoreType.DMA,
            pltpu.SemaphoreType.DMA,
        ],
    )
    def k(cc_hbm, neg_hbm, ine_hbm, oute_hbm,
          pos_out, neg_out,
          cc_i, neg_i, cen_r, ctx_r, neg_r, pos_p, neg_p, sem, sem2):
        wid = lax.axis_index("s") * _NC + lax.axis_index("c")
        pltpu.sync_copy(cc_hbm.at[wid, 0], cc_i)
        pltpu.sync_copy(neg_hbm.at[wid * _NR, 0],
                        neg_i.at[pl.ds(0, 2 * _NEG_CH)])

        def round_body(r, carry):
            c = wid * _NR + r
            cbase = r * 128
            nbase = pl.multiple_of((r % 2) * (2 * _NEG_CH), 128)
            pbase = pl.multiple_of(((r + 1) % 2) * (2 * _NEG_CH), 128)

            @pl.when(r + 1 < _NR)
            def _():
                pltpu.async_copy(neg_hbm.at[c + 1, 0],
                                 neg_i.at[pl.ds(pbase, 2 * _NEG_CH)], sem2)

            cp1 = pltpu.async_copy(
                ine_hbm.at[cc_i.at[pl.ds(cbase, 32)]], cen_r, sem)
            cp2 = pltpu.async_copy(
                oute_hbm.at[cc_i.at[pl.ds(cbase + 64, 32)]], ctx_r, sem)
            cps = [
                pltpu.async_copy(
                    oute_hbm.at[neg_i.at[pl.ds(nbase + j * 128, 128)]],
                    neg_r.at[pl.ds(j * 128, 128)], sem)
                for j in range(_NIDX_ROWS)
            ]
            cp1.wait()
            cp2.wait()
            for cp in cps:
                cp.wait()

            def b_body(b, carry2):
                oc = cc_i[pl.ds(cbase + 32 + b, 16)][0]
                ox = cc_i[pl.ds(cbase + 96 + b, 16)][0]
                c0 = cen_r[b, pl.ds(oc, 16)]
                c1 = cen_r[b, pl.ds(oc + 16, 16)]
                c2 = cen_r[b, pl.ds(oc + 32, 16)]
                c3 = cen_r[b, pl.ds(oc + 48, 16)]
                x0 = ctx_r[b, pl.ds(ox, 16)]
                x1 = ctx_r[b, pl.ds(ox + 16, 16)]
                x2 = ctx_r[b, pl.ds(ox + 32, 16)]
                x3 = ctx_r[b, pl.ds(ox + 48, 16)]
                m = r * _CH + b
                pos_p[m // 8, pl.ds((m % 8) * 16, 16)] = (
                    c0 * x0 + c1 * x1 + c2 * x2 + c3 * x3)
                nb = b * _K
                for kk in range(_K):
                    n = nb + kk
                    on = neg_i[pl.ds(nbase + _NEG_CH + n, 16)][0]
                    n0 = neg_r[n, pl.ds(on, 16)]
                    n1 = neg_r[n, pl.ds(on + 16, 16)]
                    n2 = neg_r[n, pl.ds(on + 32, 16)]
                    n3 = neg_r[n, pl.ds(on + 48, 16)]
                    neg_p[n // 8, pl.ds((n % 8) * 16, 16)] = (
                        c0 * n0 + c1 * n1 + c2 * n2 + c3 * n3)
                return carry2

            lax.fori_loop(0, _CH, b_body, 0)
            cpo = pltpu.async_copy(
                neg_p, neg_out.at[pl.ds(c * (_NEG_CH // 8), _NEG_CH // 8)],
                sem)
            cpo.wait()

            @pl.when(r + 1 < _NR)
            def _():
                pltpu.make_async_copy(
                    neg_hbm.at[c + 1, 0],
                    neg_i.at[pl.ds(pbase, 2 * _NEG_CH)], sem2).wait()

            return carry

        lax.fori_loop(0, _NR, round_body, 0)
        cpp = pltpu.async_copy(
            pos_p, pos_out.at[pl.ds(wid * (_BPW // 8), _BPW // 8)], sem)
        cpp.wait()

    return k(cc, negio, ine2, oute2)


def _tc_loss(pos_p, neg_p):
    grid = 16
    pos_blk = _POS_ROWS // grid
    neg_blk = _NEG_ROWS // grid

    def body(p_ref, n_ref, o_ref):
        sel = (jax.lax.broadcasted_iota(jnp.int32, (128, 8), 0) // 16
               == jax.lax.broadcasted_iota(jnp.int32, (128, 8), 1)
               ).astype(jnp.float32)
        ps = jnp.dot(p_ref[...], sel, preferred_element_type=jnp.float32)
        ns = jnp.dot(n_ref[...], sel, preferred_element_type=jnp.float32)
        lsp = jnp.minimum(ps, 0.0) - jnp.log1p(jnp.exp(-jnp.abs(ps)))
        lsn = jnp.minimum(-ns, 0.0) - jnp.log1p(jnp.exp(-jnp.abs(ns)))
        partial = jnp.sum(lsp) + jnp.sum(lsn)

        @pl.when(pl.program_id(0) == 0)
        def _():
            o_ref[...] = jnp.zeros((1, 1), jnp.float32)

        o_ref[...] += jnp.reshape(-partial / _B, (1, 1))

    out = pl.pallas_call(
        body,
        grid=(grid,),
        in_specs=[
            pl.BlockSpec((pos_blk, 128), lambda i: (i, 0)),
            pl.BlockSpec((neg_blk, 128), lambda i: (i, 0)),
        ],
        out_specs=pl.BlockSpec((1, 1), lambda i: (0, 0)),
        out_shape=jax.ShapeDtypeStruct((1, 1), jnp.float32),
    )(pos_p, neg_p)
    return out[0, 0]


def kernel(centers, contexts, negatives, in_embed, out_embed):
    cen = centers.astype(jnp.int32)
    ctx = contexts.astype(jnp.int32)
    neg = negatives.astype(jnp.int32).reshape(_B * _K)
    cc = jnp.concatenate(
        [(cen >> 1).reshape(_NCHUNK, _CH),
         ((cen & 1) * _D).reshape(_NCHUNK, _CH),
         (ctx >> 1).reshape(_NCHUNK, _CH),
         ((ctx & 1) * _D).reshape(_NCHUNK, _CH)],
        axis=1).reshape(_NW, 1, _NR * 128)
    negio = jnp.concatenate(
        [(neg >> 1).reshape(_NCHUNK, _NEG_CH),
         ((neg & 1) * _D).reshape(_NCHUNK, _NEG_CH)],
        axis=1).reshape(_NCHUNK, 1, 2 * _NEG_CH)
    ine2 = _sc_compact(in_embed)
    oute2 = _sc_compact(out_embed)
    pos_p, neg_p = _sc_scores(cc, negio, ine2, oute2)
    return _tc_loss(pos_p, neg_p)


# speedup vs baseline: 1.3285x; 1.3285x over previous
import functools

import jax
import jax.numpy as jnp
from jax import lax
from jax.experimental import pallas as pl
from jax.experimental.pallas import tpu as pltpu
from jax.experimental.pallas import tpu_sc as plsc

_VOCAB = 1000000
_D = 64
_B = 16384
_K = 20
_NC = 2
_NS = 16
_NW = _NC * _NS
_BPW = _B // _NW
_CH = 32
_NR = _BPW // _CH
_NCHUNK = _NW * _NR
_NEG_CH = _CH * _K
_NIDX_ROWS = _NEG_CH // 128
_POS_ROWS = _B * 16 // 128
_NEG_ROWS = _B * _K * 16 // 128


def _sc_scores(cc, negio, ine2, oute2):
    mesh = plsc.VectorSubcoreMesh(core_axis_name="c", subcore_axis_name="s")

    @functools.partial(
        pl.kernel,
        mesh=mesh,
        out_type=[
            jax.ShapeDtypeStruct((_POS_ROWS, 128), jnp.float32),
            jax.ShapeDtypeStruct((_NEG_ROWS, 128), jnp.float32),
        ],
        scratch_types=[
            pltpu.VMEM((_NR * 128,), jnp.int32),
            pltpu.VMEM((2 * 2 * _NEG_CH + 16,), jnp.int32),
            pltpu.VMEM((_CH, 128), jnp.float32),
            pltpu.VMEM((_CH, 128), jnp.float32),
            pltpu.VMEM((_NEG_CH, 128), jnp.float32),
            pltpu.VMEM((_BPW // 8, 128), jnp.float32),
            pltpu.VMEM((2 * (_NEG_CH // 8), 128), jnp.float32),
            pltpu.SemaphoreType.DMA,
            pltpu.SemaphoreType.DMA,
            pltpu.SemaphoreType.DMA,
        ],
    )
    def k(cc_hbm, neg_hbm, ine_hbm, oute_hbm,
          pos_out, neg_out,
          cc_i, neg_i, cen_r, ctx_r, neg_r, pos_p, neg_p, sem, sem2, sem3):
        wid = lax.axis_index("s") * _NC + lax.axis_index("c")
        pltpu.sync_copy(cc_hbm.at[wid, 0], cc_i)
        pltpu.sync_copy(neg_hbm.at[wid * _NR, 0],
                        neg_i.at[pl.ds(0, 2 * _NEG_CH)])

        def round_body(r, carry):
            c = wid * _NR + r
            cbase = r * 128
            nbase = pl.multiple_of((r % 2) * (2 * _NEG_CH), 128)
            pbase = pl.multiple_of(((r + 1) % 2) * (2 * _NEG_CH), 128)

            @pl.when(r + 1 < _NR)
            def _():
                pltpu.async_copy(neg_hbm.at[c + 1, 0],
                                 neg_i.at[pl.ds(pbase, 2 * _NEG_CH)], sem2)

            cp1 = pltpu.async_copy(
                ine_hbm.at[cc_i.at[pl.ds(cbase, 32)]], cen_r, sem)
            cp2 = pltpu.async_copy(
                oute_hbm.at[cc_i.at[pl.ds(cbase + 64, 32)]], ctx_r, sem)
            cps = [
                pltpu.async_copy(
                    oute_hbm.at[neg_i.at[pl.ds(nbase + j * 128, 128)]],
                    neg_r.at[pl.ds(j * 128, 128)], sem)
                for j in range(_NIDX_ROWS)
            ]
            cp1.wait()
            cp2.wait()
            for cp in cps[:3]:
                cp.wait()

            nboff = pl.multiple_of((r % 2) * (_NEG_CH // 8), 8)

            @pl.when(r >= 2)
            def _():
                pltpu.make_async_copy(
                    neg_p.at[pl.ds(nboff, _NEG_CH // 8)],
                    neg_out.at[pl.ds(0, _NEG_CH // 8)], sem3).wait()

            def b_body(b, carry2):
                oc = cc_i[pl.ds(cbase + 32 + b, 16)][0]
                ox = cc_i[pl.ds(cbase + 96 + b, 16)][0]
                c0 = cen_r[b, pl.ds(oc, 16)]
                c1 = cen_r[b, pl.ds(oc + 16, 16)]
                c2 = cen_r[b, pl.ds(oc + 32, 16)]
                c3 = cen_r[b, pl.ds(oc + 48, 16)]
                x0 = ctx_r[b, pl.ds(ox, 16)]
                x1 = ctx_r[b, pl.ds(ox + 16, 16)]
                x2 = ctx_r[b, pl.ds(ox + 32, 16)]
                x3 = ctx_r[b, pl.ds(ox + 48, 16)]
                m = r * _CH + b
                pos_p[m // 8, pl.ds((m % 8) * 16, 16)] = (
                    c0 * x0 + c1 * x1 + c2 * x2 + c3 * x3)
                nb = b * _K
                for kk in range(_K):
                    n = nb + kk
                    on = neg_i[pl.ds(nbase + _NEG_CH + n, 16)][0]
                    n0 = neg_r[n, pl.ds(on, 16)]
                    n1 = neg_r[n, pl.ds(on + 16, 16)]
                    n2 = neg_r[n, pl.ds(on + 32, 16)]
                    n3 = neg_r[n, pl.ds(on + 48, 16)]
                    neg_p[nboff + n // 8, pl.ds((n % 8) * 16, 16)] = (
                        c0 * n0 + c1 * n1 + c2 * n2 + c3 * n3)
                return carry2

            lax.fori_loop(0, 19, b_body, 0)
            for cp in cps[3:]:
                cp.wait()
            lax.fori_loop(19, _CH, b_body, 0)
            pltpu.async_copy(
                neg_p.at[pl.ds(nboff, _NEG_CH // 8)],
                neg_out.at[pl.ds(c * (_NEG_CH // 8), _NEG_CH // 8)], sem3)

            @pl.when(r + 1 < _NR)
            def _():
                pltpu.make_async_copy(
                    neg_hbm.at[c + 1, 0],
                    neg_i.at[pl.ds(pbase, 2 * _NEG_CH)], sem2).wait()

            return carry

        lax.fori_loop(0, _NR, round_body, 0)
        for _i in range(2):
            pltpu.make_async_copy(
                neg_p.at[pl.ds(0, _NEG_CH // 8)],
                neg_out.at[pl.ds(0, _NEG_CH // 8)], sem3).wait()
        cpp = pltpu.async_copy(
            pos_p, pos_out.at[pl.ds(wid * (_BPW // 8), _BPW // 8)], sem)
        cpp.wait()

    return k(cc, negio, ine2, oute2)


def _tc_loss(pos_p, neg_p):
    grid = 16
    pos_blk = _POS_ROWS // grid
    neg_blk = _NEG_ROWS // grid

    def body(p_ref, n_ref, o_ref):
        sel = (jax.lax.broadcasted_iota(jnp.int32, (128, 8), 0) // 16
               == jax.lax.broadcasted_iota(jnp.int32, (128, 8), 1)
               ).astype(jnp.float32)
        ps = jnp.dot(p_ref[...], sel, preferred_element_type=jnp.float32)
        ns = jnp.dot(n_ref[...], sel, preferred_element_type=jnp.float32)
        lsp = jnp.minimum(ps, 0.0) - jnp.log1p(jnp.exp(-jnp.abs(ps)))
        lsn = jnp.minimum(-ns, 0.0) - jnp.log1p(jnp.exp(-jnp.abs(ns)))
        partial = jnp.sum(lsp) + jnp.sum(lsn)

        @pl.when(pl.program_id(0) == 0)
        def _():
            o_ref[...] = jnp.zeros((1, 1), jnp.float32)

        o_ref[...] += jnp.reshape(-partial / _B, (1, 1))

    out = pl.pallas_call(
        body,
        grid=(grid,),
        in_specs=[
            pl.BlockSpec((pos_blk, 128), lambda i: (i, 0)),
            pl.BlockSpec((neg_blk, 128), lambda i: (i, 0)),
        ],
        out_specs=pl.BlockSpec((1, 1), lambda i: (0, 0)),
        out_shape=jax.ShapeDtypeStruct((1, 1), jnp.float32),
    )(pos_p, neg_p)
    return out[0, 0]


def kernel(centers, contexts, negatives, in_embed, out_embed):
    cen = centers.astype(jnp.int32)
    ctx = contexts.astype(jnp.int32)
    neg = negatives.astype(jnp.int32).reshape(_B * _K)
    cc = jnp.concatenate(
        [(cen >> 1).reshape(_NCHUNK, _CH),
         ((cen & 1) * _D).reshape(_NCHUNK, _CH),
         (ctx >> 1).reshape(_NCHUNK, _CH),
         ((ctx & 1) * _D).reshape(_NCHUNK, _CH)],
        axis=1).reshape(_NW, 1, _NR * 128)
    negio = jnp.concatenate(
        [(neg >> 1).reshape(_NCHUNK, _NEG_CH),
         ((neg & 1) * _D).reshape(_NCHUNK, _NEG_CH)],
        axis=1).reshape(_NCHUNK, 1, 2 * _NEG_CH)
    ine2 = in_embed.reshape(_VOCAB // 2, 128)
    oute2 = out_embed.reshape(_VOCAB // 2, 128)
    pos_p, neg_p = _sc_scores(cc, negio, ine2, oute2)
    return _tc_loss(pos_p, neg_p)
